# Initial kernel scaffold; baseline (speedup 1.0000x reference)
import functools

import jax
import jax.numpy as jnp
import numpy as np
from jax import lax
from jax.experimental import pallas as pl
from jax.experimental.pallas import tpu as pltpu
from jax.experimental.pallas import tpu_sc as plsc

B = 16384
F = 26
V = 1000
C = 1000
D = 16
DN = 13
H1 = 128
H2 = 64
EPS = 1e-5

CP = 1024
NW = 32
BPW = B // NW
NB = 4
NCH = BPW // NB
LANES = 16


def _sc_body(idx_hbm, tw_hbm, td_hbm, wide_hbm, deep_hbm,
             idx_v, gbuf, wbuf, dbuf, sem):
    wid = lax.axis_index("s") * 2 + lax.axis_index("c")

    def chunk(i, carry):
        base = wid * BPW + i * NB
        ib = base * F
        pltpu.sync_copy(idx_hbm.at[pl.ds(ib, NB * F)], idx_v)
        pltpu.async_copy(tw_hbm.at[idx_v], gbuf, sem).wait()
        pltpu.async_copy(td_hbm.at[idx_v], dbuf, sem).wait()
        for r in range(NB):
            def cc_body(cc, c2, r=r):
                o = cc * LANES
                acc = gbuf[r * F, pl.ds(o, LANES)]
                for f in range(1, F):
                    acc = acc + gbuf[r * F + f, pl.ds(o, LANES)]
                wbuf[r, pl.ds(o, LANES)] = acc
                return c2
            lax.fori_loop(0, CP // LANES, cc_body, 0)
        pltpu.sync_copy(wbuf, wide_hbm.at[pl.ds(base, NB)])
        pltpu.sync_copy(dbuf, deep_hbm.at[pl.ds(ib, NB * F)])
        return carry

    lax.fori_loop(0, NCH, chunk, 0)


def _sc_gather(idx_flat, tw, td):
    mesh = plsc.VectorSubcoreMesh(core_axis_name="c", subcore_axis_name="s")
    fn = pl.kernel(
        _sc_body,
        out_type=[
            jax.ShapeDtypeStruct((B, CP), jnp.float32),
            jax.ShapeDtypeStruct((B * F, D), jnp.float32),
        ],
        mesh=mesh,
        scratch_types=[
            pltpu.VMEM((NB * F,), jnp.int32),
            pltpu.VMEM((NB * F, CP), jnp.float32),
            pltpu.VMEM((NB, CP), jnp.float32),
            pltpu.VMEM((NB * F, D), jnp.float32),
            pltpu.SemaphoreType.DMA,
        ],
    )
    return fn(idx_flat, tw, td)


def _tc_body(wide_ref, demb_ref, xd_ref, wd_ref, bd_ref,
             w1a_ref, w1b_ref, b1_ref, g1_ref, beta1_ref,
             w2_ref, b2_ref, g2_ref, beta2_ref, w3_ref, b3_ref, out_ref):
    s = np.float32(1.0 / np.sqrt(1.0 + EPS))
    demb = demb_ref[...]
    xd = xd_ref[...]
    h = (jnp.dot(demb, w1a_ref[...], preferred_element_type=jnp.float32)
         + jnp.dot(xd, w1b_ref[...], preferred_element_type=jnp.float32)
         + b1_ref[...])
    h = g1_ref[...] * (h * s) + beta1_ref[...]
    h = jnp.maximum(h, 0.0)
    h = jnp.dot(h, w2_ref[...], preferred_element_type=jnp.float32) + b2_ref[...]
    h = g2_ref[...] * (h * s) + beta2_ref[...]
    h = jnp.maximum(h, 0.0)
    dl = jnp.dot(h, w3_ref[...], preferred_element_type=jnp.float32) + b3_ref[...]
    wl = (wide_ref[:, 0:C]
          + jnp.dot(xd, wd_ref[...], preferred_element_type=jnp.float32)
          + bd_ref[...])
    out_ref[...] = wl + dl


def _tc_mlp(wide, demb, xd, wd, bd, w1a, w1b, b1, g1, beta1,
            w2, b2, g2, beta2, w3, b3):
    TB = 1024
    grid = (B // TB,)

    def row(i):
        return (i, 0)

    def rep(i):
        return (0, 0)

    full = lambda shp: pl.BlockSpec(shp, rep)
    return pl.pallas_call(
        _tc_body,
        grid=grid,
        in_specs=[
            pl.BlockSpec((TB, CP), row),
            pl.BlockSpec((TB, F * D), row),
            pl.BlockSpec((TB, DN), row),
            full((DN, C)), full((1, C)),
            full((F * D, H1)), full((DN, H1)), full((1, H1)), full((1, H1)), full((1, H1)),
            full((H1, H2)), full((1, H2)), full((1, H2)), full((1, H2)),
            full((H2, C)), full((1, C)),
        ],
        out_specs=pl.BlockSpec((TB, C), row),
        out_shape=jax.ShapeDtypeStruct((B, C), jnp.float32),
    )(wide, demb, xd, wd, bd, w1a, w1b, b1, g1, beta1,
      w2, b2, g2, beta2, w3, b3)


def kernel(X_sparse, X_dense, wide_tables, deep_tables, Wd, bd,
           W1, b1, g1, beta1, W2, b2, g2, beta2, W3, b3):
    idx = (X_sparse.astype(jnp.int32)
           + (jnp.arange(F, dtype=jnp.int32) * V)[None, :]).reshape(-1)
    tw = jnp.pad(wide_tables.reshape(F * V, C), ((0, 0), (0, CP - C)))
    td = deep_tables.reshape(F * V, D)
    wide_acc, demb26 = _sc_gather(idx, tw, td)
    demb = demb26.reshape(B, F * D)
    r2 = lambda v: v.reshape(1, -1)
    return _tc_mlp(wide_acc, demb, X_dense, Wd, r2(bd),
                   W1[:F * D], W1[F * D:], r2(b1), r2(g1), r2(beta1),
                   W2, r2(b2), r2(g2), r2(beta2), W3, r2(b3))


# speedup vs baseline: 5.4515x; 5.4515x over previous
import functools

import jax
import jax.numpy as jnp
import numpy as np
from jax import lax
from jax.experimental import pallas as pl
from jax.experimental.pallas import tpu as pltpu
from jax.experimental.pallas import tpu_sc as plsc

B = 16384
F = 26
V = 1000
C = 1000
D = 16
DN = 13
H1 = 128
H2 = 64
EPS = 1e-5

CP = 1024
NW = 32
BPW = B // NW
NB = 4
NCH = BPW // NB
LANES = 16


def _sc_body(idx_hbm, tw_hbm, wide_hbm, deep_hbm,
             idx_v, gbuf, wbuf, dbuf, sem):
    wid = lax.axis_index("s") * 2 + lax.axis_index("c")

    def chunk(i, carry):
        base = wid * BPW + i * NB
        ib = base * F
        pltpu.sync_copy(idx_hbm.at[pl.ds(ib, NB * F)], idx_v)
        pltpu.async_copy(tw_hbm.at[idx_v], gbuf, sem).wait()
        for r in range(NB):
            def cc_body(cc, c2, r=r):
                o = cc * LANES
                acc = gbuf[r * F, pl.ds(o, LANES)]
                for f in range(1, F):
                    acc = acc + gbuf[r * F + f, pl.ds(o, LANES)]
                wbuf[r, pl.ds(o, LANES)] = acc
                return c2
            lax.fori_loop(0, CP // LANES, cc_body, 0)
        for k in range(NB * F):
            dbuf[pl.ds(k * D, D)] = gbuf[k, pl.ds(0, D)]
        pltpu.sync_copy(wbuf, wide_hbm.at[pl.ds(base, NB)])
        pltpu.sync_copy(dbuf, deep_hbm.at[pl.ds(ib * D, NB * F * D)])
        return carry

    lax.fori_loop(0, NCH, chunk, 0)


def _sc_gather(idx_flat, tw):
    mesh = plsc.VectorSubcoreMesh(core_axis_name="c", subcore_axis_name="s")
    fn = pl.kernel(
        _sc_body,
        out_type=[
            jax.ShapeDtypeStruct((B, CP), jnp.float32),
            jax.ShapeDtypeStruct((B * F * D,), jnp.float32),
        ],
        mesh=mesh,
        scratch_types=[
            pltpu.VMEM((NB * F,), jnp.int32),
            pltpu.VMEM((NB * F, CP), jnp.float32),
            pltpu.VMEM((NB, CP), jnp.float32),
            pltpu.VMEM((NB * F * D,), jnp.float32),
            pltpu.SemaphoreType.DMA,
        ],
    )
    return fn(idx_flat, tw)


def _tc_body(wide_ref, demb_ref, xd_ref, wd_ref, bd_ref,
             w1a_ref, w1b_ref, b1_ref, g1_ref, beta1_ref,
             w2_ref, b2_ref, g2_ref, beta2_ref, w3_ref, b3_ref, out_ref):
    s = np.float32(1.0 / np.sqrt(1.0 + EPS))
    demb = demb_ref[...]
    xd = xd_ref[...]
    h = (jnp.dot(demb, w1a_ref[...], preferred_element_type=jnp.float32)
         + jnp.dot(xd, w1b_ref[...], preferred_element_type=jnp.float32)
         + b1_ref[...])
    h = g1_ref[...] * (h * s) + beta1_ref[...]
    h = jnp.maximum(h, 0.0)
    h = jnp.dot(h, w2_ref[...], preferred_element_type=jnp.float32) + b2_ref[...]
    h = g2_ref[...] * (h * s) + beta2_ref[...]
    h = jnp.maximum(h, 0.0)
    dl = jnp.dot(h, w3_ref[...], preferred_element_type=jnp.float32) + b3_ref[...]
    wl = (wide_ref[:, D:D + C]
          + jnp.dot(xd, wd_ref[...], preferred_element_type=jnp.float32)
          + bd_ref[...])
    out_ref[...] = wl + dl


def _tc_mlp(wide, demb, xd, wd, bd, w1a, w1b, b1, g1, beta1,
            w2, b2, g2, beta2, w3, b3):
    TB = 1024
    grid = (B // TB,)

    def row(i):
        return (i, 0)

    def rep(i):
        return (0, 0)

    full = lambda shp: pl.BlockSpec(shp, rep)
    return pl.pallas_call(
        _tc_body,
        grid=grid,
        in_specs=[
            pl.BlockSpec((TB, CP), row),
            pl.BlockSpec((TB, F * D), row),
            pl.BlockSpec((TB, DN), row),
            full((DN, C)), full((1, C)),
            full((F * D, H1)), full((DN, H1)), full((1, H1)), full((1, H1)), full((1, H1)),
            full((H1, H2)), full((1, H2)), full((1, H2)), full((1, H2)),
            full((H2, C)), full((1, C)),
        ],
        out_specs=pl.BlockSpec((TB, C), row),
        out_shape=jax.ShapeDtypeStruct((B, C), jnp.float32),
    )(wide, demb, xd, wd, bd, w1a, w1b, b1, g1, beta1,
      w2, b2, g2, beta2, w3, b3)


def kernel(X_sparse, X_dense, wide_tables, deep_tables, Wd, bd,
           W1, b1, g1, beta1, W2, b2, g2, beta2, W3, b3):
    idx = (X_sparse.astype(jnp.int32)
           + (jnp.arange(F, dtype=jnp.int32) * V)[None, :]).reshape(-1)
    tw = jnp.concatenate(
        [deep_tables.reshape(F * V, D), wide_tables.reshape(F * V, C),
         jnp.zeros((F * V, CP - C - D), jnp.float32)], axis=1)
    wide_acc, demb_flat = _sc_gather(idx, tw)
    demb = demb_flat.reshape(B, F * D)
    r2 = lambda v: v.reshape(1, -1)
    return _tc_mlp(wide_acc, demb, X_dense, Wd, r2(bd),
                   W1[:F * D], W1[F * D:], r2(b1), r2(g1), r2(beta1),
                   W2, r2(b2), r2(g2), r2(beta2), W3, r2(b3))


# speedup vs baseline: 5.4932x; 1.0076x over previous
import functools

import jax
import jax.numpy as jnp
import numpy as np
from jax import lax
from jax.experimental import pallas as pl
from jax.experimental.pallas import tpu as pltpu
from jax.experimental.pallas import tpu_sc as plsc

B = 16384
F = 26
V = 1000
C = 1000
D = 16
DN = 13
H1 = 128
H2 = 64
EPS = 1e-5

CP = 1024
CPW = CP // 2
DL = 32
DLW = DL // 2
NW = 32
BPW = B // NW
NB = 4
NBF = NB * F
NCH = BPW // NB
BL = 32


def _sc_body(idx_hbm, tw_hbm, wide_hbm, deep_hbm,
             ibuf0, ibuf1, gbuf0, gbuf1, wbuf0, wbuf1, dbuf0, dbuf1,
             isem0, isem1, gsem0, gsem1, osem0, osem1):
    wid = lax.axis_index("s") * 2 + lax.axis_index("c")
    tb = wid * BPW
    ibufs = (ibuf0, ibuf1)
    gbufs = (gbuf0, gbuf1)
    wbufs = (wbuf0, wbuf1)
    dbufs = (dbuf0, dbuf1)
    isems = (isem0, isem1)
    gsems = (gsem0, gsem1)
    osems = (osem0, osem1)

    def idx_cp(i, b):
        return pltpu.make_async_copy(
            idx_hbm.at[pl.ds((tb + i * NB) * F, NBF)], ibufs[b], isems[b])

    def gather(i, b):
        return pltpu.make_async_copy(tw_hbm.at[ibufs[b]], gbufs[b], gsems[b])

    def wide_out(i, b):
        return pltpu.make_async_copy(
            wbufs[b], wide_hbm.at[pl.ds(tb + i * NB, NB)], osems[b])

    def deep_out(i, b):
        return pltpu.make_async_copy(
            dbufs[b],
            deep_hbm.at[pl.ds((tb + i * NB) * F * DLW, NBF * DLW)], osems[b])

    pltpu.sync_copy(idx_hbm.at[pl.ds(tb * F, NBF)], ibuf0)
    gather(0, 0).start()
    pltpu.sync_copy(idx_hbm.at[pl.ds((tb + NB) * F, NBF)], ibuf1)
    gather(1, 1).start()

    def half(j, b):
        i = j * 2 + b
        gbuf, wbuf, dbuf = gbufs[b], wbufs[b], dbufs[b]
        @pl.when(j >= 1)
        def _():
            wide_out(i, b).wait()
            deep_out(i, b).wait()
        gather(i, b).wait()
        @pl.when(i + 2 < NCH)
        def _():
            idx_cp(i + 2, b).start()
        hi = np.int32(-65536)
        for r in range(NB):
            def cc_body(cc, c2, r=r):
                o = cc * 16
                bc = lax.bitcast_convert_type
                g = gbuf[r * F, pl.ds(o, 16)]
                ae = bc(g << 16, jnp.float32)
                ao = bc(g & hi, jnp.float32)
                for f in range(1, F):
                    g = gbuf[r * F + f, pl.ds(o, 16)]
                    ae = ae + bc(g << 16, jnp.float32)
                    ao = ao + bc(g & hi, jnp.float32)
                we = lax.shift_right_logical(bc(ae, jnp.int32), 16)
                wo = bc(ao, jnp.int32) & hi
                wbuf[r, pl.ds(o, 16)] = we | wo
                return c2
            lax.fori_loop(0, CPW // 16, cc_body, 0)
        for k in range(NBF):
            dbuf[pl.ds(k * DLW, DLW)] = gbuf[k, pl.ds(0, DLW)]
        wide_out(i, b).start()
        deep_out(i, b).start()
        @pl.when(i + 2 < NCH)
        def _():
            idx_cp(i + 2, b).wait()
            gather(i + 2, b).start()

    def body(j, carry):
        half(j, 0)
        half(j, 1)
        return carry

    lax.fori_loop(0, NCH // 2, body, 0)
    wide_out(NCH - 2, 0).wait()
    deep_out(NCH - 2, 0).wait()
    wide_out(NCH - 1, 1).wait()
    deep_out(NCH - 1, 1).wait()


def _sc_gather(idx_flat, tw):
    mesh = plsc.VectorSubcoreMesh(core_axis_name="c", subcore_axis_name="s")
    fn = pl.kernel(
        _sc_body,
        out_type=[
            jax.ShapeDtypeStruct((B, CPW), jnp.int32),
            jax.ShapeDtypeStruct((B * F * DLW,), jnp.int32),
        ],
        mesh=mesh,
        scratch_types=[
            pltpu.VMEM((NBF,), jnp.int32),
            pltpu.VMEM((NBF,), jnp.int32),
            pltpu.VMEM((NBF, CPW), jnp.int32),
            pltpu.VMEM((NBF, CPW), jnp.int32),
            pltpu.VMEM((NB, CPW), jnp.int32),
            pltpu.VMEM((NB, CPW), jnp.int32),
            pltpu.VMEM((NBF * DLW,), jnp.int32),
            pltpu.VMEM((NBF * DLW,), jnp.int32),
            pltpu.SemaphoreType.DMA,
            pltpu.SemaphoreType.DMA,
            pltpu.SemaphoreType.DMA,
            pltpu.SemaphoreType.DMA,
            pltpu.SemaphoreType.DMA,
            pltpu.SemaphoreType.DMA,
        ],
    )
    return fn(idx_flat, tw)


def _tc_body(wide_ref, demb_ref, xd_ref, wd_ref, bd_ref,
             w1a_ref, w1b_ref, b1_ref, g1_ref, beta1_ref,
             w2_ref, b2_ref, g2_ref, beta2_ref, w3_ref, b3_ref, out_ref):
    s = np.float32(1.0 / np.sqrt(1.0 + EPS))
    demb = demb_ref[...]
    xd = xd_ref[...]
    h = (jnp.dot(demb, w1a_ref[...], preferred_element_type=jnp.float32)
         + jnp.dot(xd, w1b_ref[...], preferred_element_type=jnp.float32)
         + b1_ref[...])
    h = g1_ref[...] * (h * s) + beta1_ref[...]
    h = jnp.maximum(h, 0.0)
    h = jnp.dot(h, w2_ref[...], preferred_element_type=jnp.float32) + b2_ref[...]
    h = g2_ref[...] * (h * s) + beta2_ref[...]
    h = jnp.maximum(h, 0.0)
    dl = jnp.dot(h, w3_ref[...], preferred_element_type=jnp.float32) + b3_ref[...]
    wl = (wide_ref[:, D:D + C].astype(jnp.float32)
          + jnp.dot(xd, wd_ref[...], preferred_element_type=jnp.float32)
          + bd_ref[...])
    out_ref[...] = wl + dl


def _tc_mlp(wide, demb, xd, wd, bd, w1a, w1b, b1, g1, beta1,
            w2, b2, g2, beta2, w3, b3):
    TB = 1024
    grid = (B // TB,)

    def row(i):
        return (i, 0)

    def rep(i):
        return (0, 0)

    full = lambda shp: pl.BlockSpec(shp, rep)
    return pl.pallas_call(
        _tc_body,
        grid=grid,
        in_specs=[
            pl.BlockSpec((TB, CP), row),
            pl.BlockSpec((TB, F * D), row),
            pl.BlockSpec((TB, DN), row),
            full((DN, C)), full((1, C)),
            full((F * D, H1)), full((DN, H1)), full((1, H1)), full((1, H1)), full((1, H1)),
            full((H1, H2)), full((1, H2)), full((1, H2)), full((1, H2)),
            full((H2, C)), full((1, C)),
        ],
        out_specs=pl.BlockSpec((TB, C), row),
        out_shape=jax.ShapeDtypeStruct((B, C), jnp.float32),
    )(wide, demb, xd, wd, bd, w1a, w1b, b1, g1, beta1,
      w2, b2, g2, beta2, w3, b3)


def kernel(X_sparse, X_dense, wide_tables, deep_tables, Wd, bd,
           W1, b1, g1, beta1, W2, b2, g2, beta2, W3, b3):
    idx = (X_sparse.astype(jnp.int32)
           + (jnp.arange(F, dtype=jnp.int32) * V)[None, :]).reshape(-1)
    tw = jnp.concatenate(
        [deep_tables.reshape(F * V, D), wide_tables.reshape(F * V, C),
         jnp.zeros((F * V, CP - C - D), jnp.float32)],
        axis=1).astype(jnp.bfloat16)
    tw = jax.lax.bitcast_convert_type(
        tw.reshape(F * V, CPW, 2), jnp.int32)
    wide_i32, demb_i32 = _sc_gather(idx, tw)
    wide_acc = jax.lax.bitcast_convert_type(
        wide_i32, jnp.bfloat16).reshape(B, CP)
    demb = jax.lax.bitcast_convert_type(
        demb_i32.reshape(B * F, DLW),
        jnp.bfloat16).reshape(B * F, DL)[:, :D].reshape(B, F * D)
    r2 = lambda v: v.reshape(1, -1)
    return _tc_mlp(wide_acc, demb, X_dense, Wd, r2(bd),
                   W1[:F * D], W1[F * D:], r2(b1), r2(g1), r2(beta1),
                   W2, r2(b2), r2(g2), r2(beta2), W3, r2(b3))


# speedup vs baseline: 6.0469x; 1.1008x over previous
import functools

import jax
import jax.numpy as jnp
import numpy as np
from jax import lax
from jax.experimental import pallas as pl
from jax.experimental.pallas import tpu as pltpu
from jax.experimental.pallas import tpu_sc as plsc

B = 16384
F = 26
V = 1000
C = 1000
D = 16
DN = 13
H1 = 128
H2 = 64
EPS = 1e-5

CP = 1024
CPW = CP // 2
DL = 32
DLW = DL // 2
NW = 32
BPW = B // NW
NB = 4
NBF = NB * F
NCH = BPW // NB


def _pack_body(wide_ref, deep_ref, out_ref):
    out_ref[:, :C] = wide_ref[...].astype(jnp.bfloat16)
    out_ref[:, C:CP - D] = jnp.zeros((wide_ref.shape[0], CP - D - C),
                                     jnp.bfloat16)
    out_ref[:, CP - D:] = deep_ref[...].astype(jnp.bfloat16)


def _pack_table(wide_r, deep_r):
    RB = 1000
    return pl.pallas_call(
        _pack_body,
        grid=(F * V // RB,),
        in_specs=[
            pl.BlockSpec((RB, C), lambda i: (i, 0)),
            pl.BlockSpec((RB, D), lambda i: (i, 0)),
        ],
        out_specs=pl.BlockSpec((RB, CP), lambda i: (i, 0)),
        out_shape=jax.ShapeDtypeStruct((F * V, CP), jnp.bfloat16),
    )(wide_r, deep_r)


def _sc_body(idx_hbm, tw_hbm, wide_hbm, deep_hbm,
             ibuf0, ibuf1, gbuf0, gbuf1, wbuf0, wbuf1, dbuf0, dbuf1,
             isem0, isem1, gsem0, gsem1, osem0, osem1):
    wid = lax.axis_index("s") * 2 + lax.axis_index("c")
    tb = wid * BPW
    ibufs = (ibuf0, ibuf1)
    gbufs = (gbuf0, gbuf1)
    wbufs = (wbuf0, wbuf1)
    dbufs = (dbuf0, dbuf1)
    isems = (isem0, isem1)
    gsems = (gsem0, gsem1)
    osems = (osem0, osem1)

    def idx_cp(i, b):
        return pltpu.make_async_copy(
            idx_hbm.at[pl.ds((tb + i * NB) * F, NBF)], ibufs[b], isems[b])

    def gather(i, b):
        return pltpu.make_async_copy(tw_hbm.at[ibufs[b]], gbufs[b], gsems[b])

    def wide_out(i, b):
        return pltpu.make_async_copy(
            wbufs[b], wide_hbm.at[pl.ds(tb + i * NB, NB)], osems[b])

    def deep_out(i, b):
        return pltpu.make_async_copy(
            dbufs[b],
            deep_hbm.at[pl.ds((tb + i * NB) * F * DLW, NBF * DLW)], osems[b])

    pltpu.sync_copy(idx_hbm.at[pl.ds(tb * F, NBF)], ibuf0)
    gather(0, 0).start()
    pltpu.sync_copy(idx_hbm.at[pl.ds((tb + NB) * F, NBF)], ibuf1)
    gather(1, 1).start()

    def half(j, b):
        i = j * 2 + b
        gbuf, wbuf, dbuf = gbufs[b], wbufs[b], dbufs[b]
        @pl.when(j >= 1)
        def _():
            wide_out(i, b).wait()
            deep_out(i, b).wait()
        gather(i, b).wait()
        @pl.when(i + 2 < NCH)
        def _():
            idx_cp(i + 2, b).start()
        hi = np.int32(-65536)
        for r in range(NB):
            def cc_body(cc, c2, r=r):
                o = cc * 16
                bc = lax.bitcast_convert_type
                g = gbuf[r * F, pl.ds(o, 16)]
                ae = bc(g << 16, jnp.float32)
                ao = bc(g & hi, jnp.float32)
                for f in range(1, F):
                    g = gbuf[r * F + f, pl.ds(o, 16)]
                    ae = ae + bc(g << 16, jnp.float32)
                    ao = ao + bc(g & hi, jnp.float32)
                we = lax.shift_right_logical(bc(ae, jnp.int32), 16)
                wo = bc(ao, jnp.int32) & hi
                wbuf[r, pl.ds(o, 16)] = we | wo
                return c2
            lax.fori_loop(0, CPW // 16, cc_body, 0)
        for k in range(NBF):
            dbuf[pl.ds(k * DLW, DLW)] = gbuf[k, pl.ds(CPW - DLW, DLW)]
        wide_out(i, b).start()
        deep_out(i, b).start()
        @pl.when(i + 2 < NCH)
        def _():
            idx_cp(i + 2, b).wait()
            gather(i + 2, b).start()

    def body(j, carry):
        half(j, 0)
        half(j, 1)
        return carry

    lax.fori_loop(0, NCH // 2, body, 0)
    wide_out(NCH - 2, 0).wait()
    deep_out(NCH - 2, 0).wait()
    wide_out(NCH - 1, 1).wait()
    deep_out(NCH - 1, 1).wait()


def _sc_gather(idx_flat, tw):
    mesh = plsc.VectorSubcoreMesh(core_axis_name="c", subcore_axis_name="s")
    fn = pl.kernel(
        _sc_body,
        out_type=[
            jax.ShapeDtypeStruct((B, CPW), jnp.int32),
            jax.ShapeDtypeStruct((B * F * DLW,), jnp.int32),
        ],
        mesh=mesh,
        scratch_types=[
            pltpu.VMEM((NBF,), jnp.int32),
            pltpu.VMEM((NBF,), jnp.int32),
            pltpu.VMEM((NBF, CPW), jnp.int32),
            pltpu.VMEM((NBF, CPW), jnp.int32),
            pltpu.VMEM((NB, CPW), jnp.int32),
            pltpu.VMEM((NB, CPW), jnp.int32),
            pltpu.VMEM((NBF * DLW,), jnp.int32),
            pltpu.VMEM((NBF * DLW,), jnp.int32),
            pltpu.SemaphoreType.DMA,
            pltpu.SemaphoreType.DMA,
            pltpu.SemaphoreType.DMA,
            pltpu.SemaphoreType.DMA,
            pltpu.SemaphoreType.DMA,
            pltpu.SemaphoreType.DMA,
        ],
    )
    return fn(idx_flat, tw)


def _tc_body(wide_ref, demb_ref, xd_ref, wd_ref, bd_ref,
             w1a_ref, w1b_ref, b1_ref, g1_ref, beta1_ref,
             w2_ref, b2_ref, g2_ref, beta2_ref, w3_ref, b3_ref, out_ref):
    s = np.float32(1.0 / np.sqrt(1.0 + EPS))
    demb = demb_ref[...]
    xd = xd_ref[...]
    h = (jnp.dot(demb, w1a_ref[...], preferred_element_type=jnp.float32)
         + jnp.dot(xd, w1b_ref[...], preferred_element_type=jnp.float32)
         + b1_ref[...])
    h = g1_ref[...] * (h * s) + beta1_ref[...]
    h = jnp.maximum(h, 0.0)
    h = jnp.dot(h, w2_ref[...], preferred_element_type=jnp.float32) + b2_ref[...]
    h = g2_ref[...] * (h * s) + beta2_ref[...]
    h = jnp.maximum(h, 0.0)
    dl = jnp.dot(h, w3_ref[...], preferred_element_type=jnp.float32) + b3_ref[...]
    wl = (wide_ref[:, :C].astype(jnp.float32)
          + jnp.dot(xd, wd_ref[...], preferred_element_type=jnp.float32)
          + bd_ref[...])
    out_ref[...] = wl + dl


def _tc_mlp(wide, demb, xd, wd, bd, w1a, w1b, b1, g1, beta1,
            w2, b2, g2, beta2, w3, b3):
    TB = 1024
    grid = (B // TB,)

    def row(i):
        return (i, 0)

    def rep(i):
        return (0, 0)

    full = lambda shp: pl.BlockSpec(shp, rep)
    return pl.pallas_call(
        _tc_body,
        grid=grid,
        in_specs=[
            pl.BlockSpec((TB, CP), row),
            pl.BlockSpec((TB, F * DL), row),
            pl.BlockSpec((TB, DN), row),
            full((DN, C)), full((1, C)),
            full((F * DL, H1)), full((DN, H1)), full((1, H1)), full((1, H1)), full((1, H1)),
            full((H1, H2)), full((1, H2)), full((1, H2)), full((1, H2)),
            full((H2, C)), full((1, C)),
        ],
        out_specs=pl.BlockSpec((TB, C), row),
        out_shape=jax.ShapeDtypeStruct((B, C), jnp.float32),
    )(wide, demb, xd, wd, bd, w1a, w1b, b1, g1, beta1,
      w2, b2, g2, beta2, w3, b3)


def kernel(X_sparse, X_dense, wide_tables, deep_tables, Wd, bd,
           W1, b1, g1, beta1, W2, b2, g2, beta2, W3, b3):
    idx = (X_sparse.astype(jnp.int32)
           + (jnp.arange(F, dtype=jnp.int32) * V)[None, :]).reshape(-1)
    tw = _pack_table(wide_tables.reshape(F * V, C),
                     deep_tables.reshape(F * V, D))
    tw = jax.lax.bitcast_convert_type(
        tw.reshape(F * V, CPW, 2), jnp.int32)
    wide_i32, demb_i32 = _sc_gather(idx, tw)
    wide_acc = jax.lax.bitcast_convert_type(
        wide_i32, jnp.bfloat16).reshape(B, CP)
    demb = jax.lax.bitcast_convert_type(
        demb_i32.reshape(B, F * DLW), jnp.bfloat16).reshape(B, F * DL)
    w1a = jnp.pad(W1[:F * D].reshape(F, D, H1),
                  ((0, 0), (DL - D, 0), (0, 0))).reshape(F * DL, H1)
    r2 = lambda v: v.reshape(1, -1)
    return _tc_mlp(wide_acc, demb, X_dense, Wd, r2(bd),
                   w1a, W1[F * D:], r2(b1), r2(g1), r2(beta1),
                   W2, r2(b2), r2(g2), r2(beta2), W3, r2(b3))


# speedup vs baseline: 13.6566x; 2.2584x over previous
import functools

import jax
import jax.numpy as jnp
import numpy as np
from jax import lax
from jax.experimental import pallas as pl
from jax.experimental.pallas import tpu as pltpu
from jax.experimental.pallas import tpu_sc as plsc

B = 16384
F = 26
V = 1000
C = 1000
D = 16
DN = 13
H1 = 128
H2 = 64
EPS = 1e-5

CP = 1024
CPW = CP // 2
DLW = 16
NW = 32
BPW = B // NW
NB = 4
NBF = NB * F
NCH = BPW // NB

_HI = np.int32(-65536)


def _rnd(u):
    return u + np.int32(0x7FFF) + (lax.shift_right_logical(u, 16) & 1)


def _pack_body(wide_ref, deep_ref, out_ref):
    rows = wide_ref.shape[0]
    lo = wide_ref[:, :CPW]
    hi = jnp.concatenate(
        [wide_ref[:, CPW:C], jnp.zeros((rows, CP - C - D), jnp.float32),
         deep_ref[...]], axis=1)
    a = _rnd(lax.bitcast_convert_type(lo, jnp.int32))
    b = _rnd(lax.bitcast_convert_type(hi, jnp.int32))
    out_ref[...] = lax.shift_right_logical(a, 16) | (b & _HI)


def _pack_table(wide_r, deep_r):
    RB = 1000
    return pl.pallas_call(
        _pack_body,
        grid=(F * V // RB,),
        in_specs=[
            pl.BlockSpec((RB, C), lambda i: (i, 0)),
            pl.BlockSpec((RB, D), lambda i: (i, 0)),
        ],
        out_specs=pl.BlockSpec((RB, CPW), lambda i: (i, 0)),
        out_shape=jax.ShapeDtypeStruct((F * V, CPW), jnp.int32),
    )(wide_r, deep_r)


def _sc_body(idx_hbm, tw_hbm, wide_hbm, deep_hbm,
             ibuf0, ibuf1, gbuf0, gbuf1, wbuf0, wbuf1, dbuf0, dbuf1,
             isem0, isem1, gsem0, gsem1, osem0, osem1):
    wid = lax.axis_index("s") * 2 + lax.axis_index("c")
    tb = wid * BPW
    ibufs = (ibuf0, ibuf1)
    gbufs = (gbuf0, gbuf1)
    wbufs = (wbuf0, wbuf1)
    dbufs = (dbuf0, dbuf1)
    isems = (isem0, isem1)
    gsems = (gsem0, gsem1)
    osems = (osem0, osem1)

    def idx_cp(i, b):
        return pltpu.make_async_copy(
            idx_hbm.at[pl.ds((tb + i * NB) * F, NBF)], ibufs[b], isems[b])

    def gather(i, b):
        return pltpu.make_async_copy(tw_hbm.at[ibufs[b]], gbufs[b], gsems[b])

    def wide_out(i, b):
        return pltpu.make_async_copy(
            wbufs[b], wide_hbm.at[pl.ds(tb + i * NB, NB)], osems[b])

    def deep_out(i, b):
        return pltpu.make_async_copy(
            dbufs[b], deep_hbm.at[pl.ds(tb + i * NB, NB)], osems[b])

    pltpu.sync_copy(idx_hbm.at[pl.ds(tb * F, NBF)], ibuf0)
    gather(0, 0).start()
    pltpu.sync_copy(idx_hbm.at[pl.ds((tb + NB) * F, NBF)], ibuf1)
    gather(1, 1).start()

    def half(j, b):
        i = j * 2 + b
        gbuf, wbuf, dbuf = gbufs[b], wbufs[b], dbufs[b]
        @pl.when(j >= 1)
        def _():
            wide_out(i, b).wait()
            deep_out(i, b).wait()
        gather(i, b).wait()
        @pl.when(i + 2 < NCH)
        def _():
            idx_cp(i + 2, b).start()
        for r in range(NB):
            def cc_body(cc, c2, r=r):
                o = cc * 16
                bc = lax.bitcast_convert_type
                g = gbuf[r * F, pl.ds(o, 16)]
                ae = bc(g << 16, jnp.float32)
                ao = bc(g & _HI, jnp.float32)
                for f in range(1, F):
                    g = gbuf[r * F + f, pl.ds(o, 16)]
                    ae = ae + bc(g << 16, jnp.float32)
                    ao = ao + bc(g & _HI, jnp.float32)
                we = lax.shift_right_logical(bc(ae, jnp.int32), 16)
                wo = bc(ao, jnp.int32) & _HI
                wbuf[r, pl.ds(o, 16)] = we | wo
                return c2
            lax.fori_loop(0, CPW // 16, cc_body, 0)
        for k in range(NBF):
            dbuf[k // F, pl.ds((k % F) * DLW, DLW)] = gbuf[k, pl.ds(CPW - DLW, DLW)]
        wide_out(i, b).start()
        deep_out(i, b).start()
        @pl.when(i + 2 < NCH)
        def _():
            idx_cp(i + 2, b).wait()
            gather(i + 2, b).start()

    def body(j, carry):
        half(j, 0)
        half(j, 1)
        return carry

    lax.fori_loop(0, NCH // 2, body, 0)
    wide_out(NCH - 2, 0).wait()
    deep_out(NCH - 2, 0).wait()
    wide_out(NCH - 1, 1).wait()
    deep_out(NCH - 1, 1).wait()


def _sc_gather(idx_flat, tw):
    mesh = plsc.VectorSubcoreMesh(core_axis_name="c", subcore_axis_name="s")
    fn = pl.kernel(
        _sc_body,
        out_type=[
            jax.ShapeDtypeStruct((B, CPW), jnp.int32),
            jax.ShapeDtypeStruct((B, F * DLW), jnp.int32),
        ],
        mesh=mesh,
        scratch_types=[
            pltpu.VMEM((NBF,), jnp.int32),
            pltpu.VMEM((NBF,), jnp.int32),
            pltpu.VMEM((NBF, CPW), jnp.int32),
            pltpu.VMEM((NBF, CPW), jnp.int32),
            pltpu.VMEM((NB, CPW), jnp.int32),
            pltpu.VMEM((NB, CPW), jnp.int32),
            pltpu.VMEM((NB, F * DLW), jnp.int32),
            pltpu.VMEM((NB, F * DLW), jnp.int32),
            pltpu.SemaphoreType.DMA,
            pltpu.SemaphoreType.DMA,
            pltpu.SemaphoreType.DMA,
            pltpu.SemaphoreType.DMA,
            pltpu.SemaphoreType.DMA,
            pltpu.SemaphoreType.DMA,
        ],
    )
    return fn(idx_flat, tw)


def _tc_body(wide_ref, demb_ref, xd_ref, wd_ref, bd_ref,
             w1a_ref, w1b_ref, b1_ref, g1_ref, beta1_ref,
             w2_ref, b2_ref, g2_ref, beta2_ref, w3_ref, b3_ref, out_ref):
    s = np.float32(1.0 / np.sqrt(1.0 + EPS))
    bc = lax.bitcast_convert_type
    demb = bc(demb_ref[...] & _HI, jnp.float32)
    xd = xd_ref[...]
    h = (jnp.dot(demb, w1a_ref[...], preferred_element_type=jnp.float32)
         + jnp.dot(xd, w1b_ref[...], preferred_element_type=jnp.float32)
         + b1_ref[...])
    h = g1_ref[...] * (h * s) + beta1_ref[...]
    h = jnp.maximum(h, 0.0)
    h = jnp.dot(h, w2_ref[...], preferred_element_type=jnp.float32) + b2_ref[...]
    h = g2_ref[...] * (h * s) + beta2_ref[...]
    h = jnp.maximum(h, 0.0)
    dl = jnp.dot(h, w3_ref[...], preferred_element_type=jnp.float32) + b3_ref[...]
    w = wide_ref[...]
    wlo = bc(w << 16, jnp.float32)
    whi = bc(w & _HI, jnp.float32)
    wide_logit = jnp.concatenate([wlo, whi[:, :C - CPW]], axis=1)
    wl = (wide_logit
          + jnp.dot(xd, wd_ref[...], preferred_element_type=jnp.float32)
          + bd_ref[...])
    out_ref[...] = wl + dl


def _tc_mlp(wide, demb, xd, wd, bd, w1a, w1b, b1, g1, beta1,
            w2, b2, g2, beta2, w3, b3):
    TB = 1024
    grid = (B // TB,)

    def row(i):
        return (i, 0)

    def rep(i):
        return (0, 0)

    full = lambda shp: pl.BlockSpec(shp, rep)
    return pl.pallas_call(
        _tc_body,
        grid=grid,
        in_specs=[
            pl.BlockSpec((TB, CPW), row),
            pl.BlockSpec((TB, F * DLW), row),
            pl.BlockSpec((TB, DN), row),
            full((DN, C)), full((1, C)),
            full((F * D, H1)), full((DN, H1)), full((1, H1)), full((1, H1)), full((1, H1)),
            full((H1, H2)), full((1, H2)), full((1, H2)), full((1, H2)),
            full((H2, C)), full((1, C)),
        ],
        out_specs=pl.BlockSpec((TB, C), row),
        out_shape=jax.ShapeDtypeStruct((B, C), jnp.float32),
    )(wide, demb, xd, wd, bd, w1a, w1b, b1, g1, beta1,
      w2, b2, g2, beta2, w3, b3)


def kernel(X_sparse, X_dense, wide_tables, deep_tables, Wd, bd,
           W1, b1, g1, beta1, W2, b2, g2, beta2, W3, b3):
    idx = (X_sparse.astype(jnp.int32)
           + (jnp.arange(F, dtype=jnp.int32) * V)[None, :]).reshape(-1)
    tw = _pack_table(wide_tables.reshape(F * V, C),
                     deep_tables.reshape(F * V, D))
    wide_i32, demb_i32 = _sc_gather(idx, tw)
    r2 = lambda v: v.reshape(1, -1)
    return _tc_mlp(wide_i32, demb_i32, X_dense,
                   Wd, r2(bd),
                   W1[:F * D], W1[F * D:], r2(b1), r2(g1), r2(beta1),
                   W2, r2(b2), r2(g2), r2(beta2), W3, r2(b3))
